# baseline (speedup 1.0000x reference)
import functools
import jax
import jax.numpy as jnp
from jax import lax
from jax.experimental import pallas as pl
from jax.experimental.pallas import tpu as pltpu
from jax.experimental.pallas import tpu_sc as plsc

_E = 16
_H = 1024
_F = 2048
_CAP = 24
_EPS = 1e-06
_N = 64
_S = _E * _CAP
_SP = 512
_NC = 2
_NS = 16
_NW = _NC * _NS

_FB = 1024
_NF = _F // _FB


def _route_kernel(x_ref, rms_ref, wr_ref, bias_ref,
                  h_ref, inv_ref, slots_ref, w_ref):
    x = x_ref[...]
    v = jnp.mean(x * x, axis=-1, keepdims=True)
    h = rms_ref[...] * (x * jax.lax.rsqrt(v + _EPS))
    h_ref[...] = jnp.zeros((_N + 8, _H), jnp.float32)
    h_ref[pl.ds(0, _N), :] = h
    logits = jnp.dot(h, wr_ref[...], preferred_element_type=jnp.float32)
    scores = jax.nn.sigmoid(logits)
    choice = scores + bias_ref[...]
    ei = jax.lax.broadcasted_iota(jnp.int32, (_N, _E), 1)
    m1 = jnp.max(choice, axis=1, keepdims=True)
    idx1 = jnp.min(jnp.where(choice == m1, ei, _E), axis=1, keepdims=True)
    choice2 = jnp.where(ei == idx1, -jnp.inf, choice)
    m2 = jnp.max(choice2, axis=1, keepdims=True)
    idx2 = jnp.min(jnp.where(choice2 == m2, ei, _E), axis=1, keepdims=True)
    w1 = jnp.sum(jnp.where(ei == idx1, scores, 0.0), axis=1, keepdims=True)
    w2 = jnp.sum(jnp.where(ei == idx2, scores, 0.0), axis=1, keepdims=True)
    ws = w1 + w2 + 1e-09
    w1 = w1 / ws
    w2 = w2 / ws
    oh = ((ei == idx1).astype(jnp.float32)
          + (ei == idx2).astype(jnp.float32))
    ti = jax.lax.broadcasted_iota(jnp.int32, (_N, _N), 0)
    tj = jax.lax.broadcasted_iota(jnp.int32, (_N, _N), 1)
    ltri = (tj < ti).astype(jnp.float32)
    cnt_before = jnp.dot(ltri, oh, preferred_element_type=jnp.float32)
    pos1 = jnp.sum(jnp.where(ei == idx1, cnt_before, 0.0),
                   axis=1, keepdims=True).astype(jnp.int32)
    pos2 = jnp.sum(jnp.where(ei == idx2, cnt_before, 0.0),
                   axis=1, keepdims=True).astype(jnp.int32)
    slot1 = idx1 * _CAP + pos1
    slot2 = idx2 * _CAP + pos2
    ok1 = pos1 < _CAP
    ok2 = pos2 < _CAP
    aj = jax.lax.broadcasted_iota(jnp.int32, (1, 2 * _N), 1)
    at = aj // 2
    is_k0 = (aj % 2) == 0
    tok_oh = (jax.lax.broadcasted_iota(jnp.int32, (2 * _N, _N), 0) // 2
              == jax.lax.broadcasted_iota(jnp.int32, (2 * _N, _N), 1)
              ).astype(jnp.float32)
    k0 = (jax.lax.broadcasted_iota(jnp.int32, (2 * _N, 1), 0) % 2) == 0
    hi = jax.lax.Precision.HIGHEST
    s1a = jnp.dot(tok_oh, slot1.astype(jnp.float32),
                  preferred_element_type=jnp.float32, precision=hi)
    s2a = jnp.dot(tok_oh, slot2.astype(jnp.float32),
                  preferred_element_type=jnp.float32, precision=hi)
    o1a = jnp.dot(tok_oh, ok1.astype(jnp.float32),
                  preferred_element_type=jnp.float32, precision=hi)
    o2a = jnp.dot(tok_oh, ok2.astype(jnp.float32),
                  preferred_element_type=jnp.float32, precision=hi)
    w1a = jnp.dot(tok_oh, w1, preferred_element_type=jnp.float32, precision=hi)
    w2a = jnp.dot(tok_oh, w2, preferred_element_type=jnp.float32, precision=hi)
    sa = jnp.where(k0, s1a, s2a)
    oa = jnp.where(k0, o1a, o2a)
    wa = jnp.where(k0, w1a, w2a)
    slots_ref[...] = jnp.where(oa > 0.5, sa, 0.0).astype(jnp.int32).reshape(1, 2 * _N)
    w_ref[...] = (wa * oa) * jnp.ones((1, 16), jnp.float32)
    sj = jax.lax.broadcasted_iota(jnp.int32, (_N, _SP), 1)
    ind = (jnp.where((sj == slot1) & ok1, 1.0, 0.0)
           + jnp.where((sj == slot2) & ok2, 1.0, 0.0))
    tid = jax.lax.broadcasted_iota(jnp.int32, (1, _N), 1).astype(jnp.float32)
    inv_f = jnp.dot(tid, ind, preferred_element_type=jnp.float32,
                    precision=hi)
    occ = jnp.dot(jnp.ones((1, _N), jnp.float32), ind,
                  preferred_element_type=jnp.float32, precision=hi)
    inv_ref[...] = (inv_f + _N * (1.0 - occ)).astype(jnp.int32)


def _route(x, rms_w, W_router, bias_corr):
    return pl.pallas_call(
        _route_kernel,
        in_specs=[
            pl.BlockSpec((_N, _H), lambda: (0, 0)),
            pl.BlockSpec((1, _H), lambda: (0, 0)),
            pl.BlockSpec((_H, _E), lambda: (0, 0)),
            pl.BlockSpec((1, _E), lambda: (0, 0)),
        ],
        out_specs=[
            pl.BlockSpec((_N + 8, _H), lambda: (0, 0)),
            pl.BlockSpec((1, _SP), lambda: (0, 0)),
            pl.BlockSpec((1, 2 * _N), lambda: (0, 0)),
            pl.BlockSpec((2 * _N, 16), lambda: (0, 0)),
        ],
        out_shape=[
            jax.ShapeDtypeStruct((_N + 8, _H), jnp.float32),
            jax.ShapeDtypeStruct((1, _SP), jnp.int32),
            jax.ShapeDtypeStruct((1, 2 * _N), jnp.int32),
            jax.ShapeDtypeStruct((2 * _N, 16), jnp.float32),
        ],
    )(x, rms_w.reshape(1, _H), W_router, bias_corr.reshape(1, _E))


def _dispatch_body(h_hbm, inv_hbm, buf_hbm, idx_v, rows_v, sem):
    wid = lax.axis_index("s") * _NC + lax.axis_index("c")
    pltpu.sync_copy(inv_hbm.at[wid], idx_v)
    pltpu.async_copy(h_hbm.at[idx_v], rows_v, sem).wait()
    pltpu.sync_copy(rows_v, buf_hbm.at[pl.ds(wid * _NS, _NS)])


def _dispatch(h_ext, inv32):
    mesh = plsc.VectorSubcoreMesh(core_axis_name="c", subcore_axis_name="s",
                                  num_cores=_NC, num_subcores=_NS)
    return pl.kernel(
        _dispatch_body,
        out_type=jax.ShapeDtypeStruct((_SP, _H), jnp.float32),
        mesh=mesh,
        scratch_types=[
            pltpu.VMEM((_NS,), jnp.int32),
            pltpu.VMEM((_NS, _H), jnp.float32),
            pltpu.SemaphoreType.DMA,
        ],
    )(h_ext, inv32)


def _mlp_kernel(buf_ref, gate_ref, up_ref, down_ref, y_ref):
    f = pl.program_id(1)
    be = buf_ref[...]
    g = jnp.dot(be, gate_ref[0], preferred_element_type=jnp.float32)
    u = jnp.dot(be, up_ref[0], preferred_element_type=jnp.float32)
    act = g * jax.nn.sigmoid(g) * u
    contrib = jnp.dot(act, down_ref[0], preferred_element_type=jnp.float32)

    @pl.when(f == 0)
    def _init():
        y_ref[...] = contrib

    @pl.when(f > 0)
    def _acc():
        y_ref[...] += contrib


def _mlp(buf, W_gate, W_up, W_down):
    return pl.pallas_call(
        _mlp_kernel,
        grid=(_E, _NF),
        in_specs=[
            pl.BlockSpec((_CAP, _H), lambda e, f: (e, 0)),
            pl.BlockSpec((1, _H, _FB), lambda e, f: (e, 0, f)),
            pl.BlockSpec((1, _H, _FB), lambda e, f: (e, 0, f)),
            pl.BlockSpec((1, _FB, _H), lambda e, f: (e, f, 0)),
        ],
        out_specs=pl.BlockSpec((_CAP, _H), lambda e, f: (e, 0)),
        out_shape=jax.ShapeDtypeStruct((_S, _H), jnp.float32),
        compiler_params=pltpu.CompilerParams(
            dimension_semantics=("arbitrary", "arbitrary")),
    )(buf, W_gate, W_up, W_down)


def _combine_body(x_hbm, y_hbm, slots_hbm, w_hbm, out_hbm,
                  idx_v, w_v, rows_v, x_v, o_v, sem):
    wid = lax.axis_index("s") * _NC + lax.axis_index("c")
    pltpu.sync_copy(slots_hbm.at[wid], idx_v)
    pltpu.sync_copy(w_hbm.at[pl.ds(wid * 4, 4)], w_v)
    pltpu.async_copy(y_hbm.at[idx_v], rows_v, sem).wait()
    pltpu.sync_copy(x_hbm.at[pl.ds(wid * 2, 2)], x_v)

    def chunk(c, _):
        for j in range(2):
            w0 = w_v[2 * j, :]
            w1 = w_v[2 * j + 1, :]
            xc = x_v[j, pl.ds(c * 16, 16)]
            r0 = rows_v[2 * j, pl.ds(c * 16, 16)]
            r1 = rows_v[2 * j + 1, pl.ds(c * 16, 16)]
            o_v[j, pl.ds(c * 16, 16)] = xc + w0 * r0 + w1 * r1
        return _

    lax.fori_loop(0, _H // 16, chunk, 0)
    pltpu.sync_copy(o_v, out_hbm.at[pl.ds(wid * 2, 2)])


def _combine(x, y, slots32, w32):
    mesh = plsc.VectorSubcoreMesh(core_axis_name="c", subcore_axis_name="s",
                                  num_cores=_NC, num_subcores=_NS)
    return pl.kernel(
        _combine_body,
        out_type=jax.ShapeDtypeStruct((_N, _H), jnp.float32),
        mesh=mesh,
        scratch_types=[
            pltpu.VMEM((16,), jnp.int32),
            pltpu.VMEM((4, 16), jnp.float32),
            pltpu.VMEM((16, _H), jnp.float32),
            pltpu.VMEM((2, _H), jnp.float32),
            pltpu.VMEM((2, _H), jnp.float32),
            pltpu.SemaphoreType.DMA,
        ],
    )(x, y, slots32, w32)


def kernel(x, rms_w, W_router, bias_corr, W_gate, W_up, W_down):
    h_ext, inv, slots, w = _route(x, rms_w, W_router, bias_corr)
    inv32 = inv.reshape(_NW, _NS)
    slots32 = jnp.pad(slots.reshape(_NW, 4), ((0, 0), (0, 12)))
    buf = _dispatch(h_ext, inv32)
    y = _mlp(buf[:_S], W_gate, W_up, W_down)
    return _combine(x, y, slots32, w)


# speedup vs baseline: 1.5237x; 1.5237x over previous
import jax
import jax.numpy as jnp
from jax.experimental import pallas as pl
from jax.experimental.pallas import tpu as pltpu

_E = 16
_K = 2
_H = 1024
_F = 2048
_CAP = 24
_EPS = 1e-06
_N = 64
_S = _E * _CAP

_FB = 1024
_NF = _F // _FB


def _moe_kernel(x_ref, rms_ref, wr_ref, bias_ref, gate_ref, up_ref, down_ref,
                out_ref, buf_scr, g_scr, y_scr):
    e = pl.program_id(0)
    f = pl.program_id(1)

    @pl.when(jnp.logical_and(e == 0, f == 0))
    def _prologue():
        x = x_ref[...]
        v = jnp.mean(x * x, axis=-1, keepdims=True)
        h = rms_ref[...] * (x * jax.lax.rsqrt(v + _EPS))
        logits = jnp.dot(h, wr_ref[...], preferred_element_type=jnp.float32)
        scores = jax.nn.sigmoid(logits)
        choice = scores + bias_ref[...]
        ei = jax.lax.broadcasted_iota(jnp.int32, (_N, _E), 1)
        m1 = jnp.max(choice, axis=1, keepdims=True)
        idx1 = jnp.min(jnp.where(choice == m1, ei, _E), axis=1, keepdims=True)
        choice2 = jnp.where(ei == idx1, -jnp.inf, choice)
        m2 = jnp.max(choice2, axis=1, keepdims=True)
        idx2 = jnp.min(jnp.where(choice2 == m2, ei, _E), axis=1, keepdims=True)
        w1 = jnp.sum(jnp.where(ei == idx1, scores, 0.0), axis=1, keepdims=True)
        w2 = jnp.sum(jnp.where(ei == idx2, scores, 0.0), axis=1, keepdims=True)
        ws = w1 + w2 + 1e-09
        w1 = w1 / ws
        w2 = w2 / ws
        oh = ((ei == idx1).astype(jnp.float32)
              + (ei == idx2).astype(jnp.float32))
        ti = jax.lax.broadcasted_iota(jnp.int32, (_N, _N), 0)
        tj = jax.lax.broadcasted_iota(jnp.int32, (_N, _N), 1)
        ltri = (tj < ti).astype(jnp.float32)
        cnt_before = jnp.dot(ltri, oh, preferred_element_type=jnp.float32)
        pos1 = jnp.sum(jnp.where(ei == idx1, cnt_before, 0.0),
                       axis=1, keepdims=True).astype(jnp.int32)
        pos2 = jnp.sum(jnp.where(ei == idx2, cnt_before, 0.0),
                       axis=1, keepdims=True).astype(jnp.int32)
        slot1 = idx1 * _CAP + pos1
        slot2 = idx2 * _CAP + pos2
        ok1 = pos1 < _CAP
        ok2 = pos2 < _CAP
        sj = jax.lax.broadcasted_iota(jnp.int32, (_N, _S), 1)
        g_scr[...] = (jnp.where((sj == slot1) & ok1, w1, 0.0)
                      + jnp.where((sj == slot2) & ok2, w2, 0.0))
        ind = (jnp.where((sj == slot1) & ok1, 1.0, 0.0)
               + jnp.where((sj == slot2) & ok2, 1.0, 0.0))
        buf_scr[...] = jax.lax.dot_general(
            ind, h, (((0,), (0,)), ((), ())),
            preferred_element_type=jnp.float32,
            precision=jax.lax.Precision.HIGHEST)

    be = buf_scr[pl.ds(e * _CAP, _CAP), :]
    g = jnp.dot(be, gate_ref[0], preferred_element_type=jnp.float32)
    u = jnp.dot(be, up_ref[0], preferred_element_type=jnp.float32)
    act = g * jax.nn.sigmoid(g) * u
    contrib = jnp.dot(act, down_ref[0], preferred_element_type=jnp.float32)

    @pl.when(f == 0)
    def _init():
        y_scr[pl.ds(e * _CAP, _CAP), :] = contrib

    @pl.when(f > 0)
    def _acc():
        y_scr[pl.ds(e * _CAP, _CAP), :] += contrib

    @pl.when(jnp.logical_and(e == _E - 1, f == _NF - 1))
    def _epilogue():
        out_ref[...] = x_ref[...] + jnp.dot(
            g_scr[...], y_scr[...], preferred_element_type=jnp.float32,
            precision=jax.lax.Precision.HIGHEST)


def kernel(x, rms_w, W_router, bias_corr, W_gate, W_up, W_down):
    return pl.pallas_call(
        _moe_kernel,
        grid=(_E, _NF),
        in_specs=[
            pl.BlockSpec((_N, _H), lambda e, f: (0, 0)),
            pl.BlockSpec((1, _H), lambda e, f: (0, 0)),
            pl.BlockSpec((_H, _E), lambda e, f: (0, 0)),
            pl.BlockSpec((1, _E), lambda e, f: (0, 0)),
            pl.BlockSpec((1, _H, _FB), lambda e, f: (e, 0, f)),
            pl.BlockSpec((1, _H, _FB), lambda e, f: (e, 0, f)),
            pl.BlockSpec((1, _FB, _H), lambda e, f: (e, f, 0)),
        ],
        out_specs=pl.BlockSpec((_N, _H), lambda e, f: (0, 0)),
        out_shape=jax.ShapeDtypeStruct((_N, _H), jnp.float32),
        scratch_shapes=[
            pltpu.VMEM((_S, _H), jnp.float32),
            pltpu.VMEM((_N, _S), jnp.float32),
            pltpu.VMEM((_S, _H), jnp.float32),
        ],
        compiler_params=pltpu.CompilerParams(
            dimension_semantics=("arbitrary", "arbitrary")),
    )(x, rms_w.reshape(1, _H), W_router, bias_corr.reshape(1, _E),
      W_gate, W_up, W_down)


# speedup vs baseline: 1.5797x; 1.0367x over previous
import jax
import jax.numpy as jnp
from jax.experimental import pallas as pl
from jax.experimental.pallas import tpu as pltpu

_E = 16
_K = 2
_H = 1024
_F = 2048
_CAP = 24
_EPS = 1e-06
_N = 64
_S = _E * _CAP

_FB = 1024
_NF = _F // _FB


def _moe_kernel(x_ref, rms_ref, wr_ref, bias_ref, gate_ref, up_ref, down_ref,
                out_ref, buf_scr, g_scr, y_scr):
    e = pl.program_id(0)
    f = pl.program_id(1)

    @pl.when(jnp.logical_and(e == 0, f == 0))
    def _prologue():
        x = x_ref[...]
        v = jnp.mean(x * x, axis=-1, keepdims=True)
        h = rms_ref[...] * (x * jax.lax.rsqrt(v + _EPS))
        logits = jnp.dot(h, wr_ref[...], preferred_element_type=jnp.float32)
        scores = jax.nn.sigmoid(logits)
        choice = scores + bias_ref[...]
        ei = jax.lax.broadcasted_iota(jnp.int32, (_N, _E), 1)
        m1 = jnp.max(choice, axis=1, keepdims=True)
        idx1 = jnp.min(jnp.where(choice == m1, ei, _E), axis=1, keepdims=True)
        choice2 = jnp.where(ei == idx1, -jnp.inf, choice)
        m2 = jnp.max(choice2, axis=1, keepdims=True)
        idx2 = jnp.min(jnp.where(choice2 == m2, ei, _E), axis=1, keepdims=True)
        w1 = jnp.sum(jnp.where(ei == idx1, scores, 0.0), axis=1, keepdims=True)
        w2 = jnp.sum(jnp.where(ei == idx2, scores, 0.0), axis=1, keepdims=True)
        ws = w1 + w2 + 1e-09
        w1 = w1 / ws
        w2 = w2 / ws
        oh = ((ei == idx1).astype(jnp.float32)
              + (ei == idx2).astype(jnp.float32))
        ti = jax.lax.broadcasted_iota(jnp.int32, (_N, _N), 0)
        tj = jax.lax.broadcasted_iota(jnp.int32, (_N, _N), 1)
        ltri = (tj < ti).astype(jnp.float32)
        cnt_before = jnp.dot(ltri, oh, preferred_element_type=jnp.float32)
        pos1 = jnp.sum(jnp.where(ei == idx1, cnt_before, 0.0),
                       axis=1, keepdims=True).astype(jnp.int32)
        pos2 = jnp.sum(jnp.where(ei == idx2, cnt_before, 0.0),
                       axis=1, keepdims=True).astype(jnp.int32)
        slot1 = idx1 * _CAP + pos1
        slot2 = idx2 * _CAP + pos2
        ok1 = pos1 < _CAP
        ok2 = pos2 < _CAP
        sj = jax.lax.broadcasted_iota(jnp.int32, (_N, _S), 1)
        g_scr[...] = (jnp.where((sj == slot1) & ok1, w1, 0.0)
                      + jnp.where((sj == slot2) & ok2, w2, 0.0))
        ind = (jnp.where((sj == slot1) & ok1, 1.0, 0.0)
               + jnp.where((sj == slot2) & ok2, 1.0, 0.0))
        buf_scr[...] = jax.lax.dot_general(
            ind, h, (((0,), (0,)), ((), ())),
            preferred_element_type=jnp.float32)

    be = buf_scr[pl.ds(e * _CAP, _CAP), :]
    g = jnp.dot(be, gate_ref[0], preferred_element_type=jnp.float32)
    u = jnp.dot(be, up_ref[0], preferred_element_type=jnp.float32)
    act = g * jax.nn.sigmoid(g) * u
    contrib = jnp.dot(act, down_ref[0], preferred_element_type=jnp.float32)

    @pl.when(f == 0)
    def _init():
        y_scr[pl.ds(e * _CAP, _CAP), :] = contrib

    @pl.when(f > 0)
    def _acc():
        y_scr[pl.ds(e * _CAP, _CAP), :] += contrib

    @pl.when(jnp.logical_and(e == _E - 1, f == _NF - 1))
    def _epilogue():
        out_ref[...] = x_ref[...] + jnp.dot(
            g_scr[...], y_scr[...], preferred_element_type=jnp.float32)


def kernel(x, rms_w, W_router, bias_corr, W_gate, W_up, W_down):
    return pl.pallas_call(
        _moe_kernel,
        grid=(_E, _NF),
        in_specs=[
            pl.BlockSpec((_N, _H), lambda e, f: (0, 0)),
            pl.BlockSpec((1, _H), lambda e, f: (0, 0)),
            pl.BlockSpec((_H, _E), lambda e, f: (0, 0)),
            pl.BlockSpec((1, _E), lambda e, f: (0, 0)),
            pl.BlockSpec((1, _H, _FB), lambda e, f: (e, 0, f)),
            pl.BlockSpec((1, _H, _FB), lambda e, f: (e, 0, f)),
            pl.BlockSpec((1, _FB, _H), lambda e, f: (e, f, 0)),
        ],
        out_specs=pl.BlockSpec((_N, _H), lambda e, f: (0, 0)),
        out_shape=jax.ShapeDtypeStruct((_N, _H), jnp.float32),
        scratch_shapes=[
            pltpu.VMEM((_S, _H), jnp.float32),
            pltpu.VMEM((_N, _S), jnp.float32),
            pltpu.VMEM((_S, _H), jnp.float32),
        ],
        compiler_params=pltpu.CompilerParams(
            dimension_semantics=("arbitrary", "arbitrary")),
    )(x, rms_w.reshape(1, _H), W_router, bias_corr.reshape(1, _E),
      W_gate, W_up, W_down)
